# baseline (speedup 1.0000x reference)
import functools

import jax
import jax.numpy as jnp
from jax import lax
from jax.experimental import pallas as pl
from jax.experimental.pallas import tpu as pltpu
from jax.experimental.pallas import tpu_sc as plsc

N = 10000
E = 320000
H = 128
G = 64
L = 3

NC, NS = 2, 16
NW = NC * NS
CHUNK = 128
CPW = 80
EPW = CPW * CHUNK
NBUF = 2
DSTAGE = 40
EPAD = EPW * NW
NPAD = 10112
ZR = NPAD // NS

BN = 1000
NB = N // BN

_PREC = lax.Precision.DEFAULT

_sc_mesh = plsc.VectorSubcoreMesh(core_axis_name="c", subcore_axis_name="s")


@functools.partial(
    pl.kernel,
    out_type=jax.ShapeDtypeStruct((NC, NPAD, H), jnp.float32),
    mesh=_sc_mesh,
    scratch_types=[
        pltpu.VMEM((DSTAGE, CHUNK), jnp.int32),
        pltpu.VMEM((DSTAGE, CHUNK), jnp.int32),
        pltpu.VMEM((NBUF, CHUNK, H), jnp.float32),
        pltpu.VMEM_SHARED((NPAD, H), jnp.float32),
        pltpu.SemaphoreType.DMA((NBUF,)),
    ],
)
def _sc_edge_aggregate(m_hbm, srcs_hbm, dsts_hbm, zeros_hbm, out_hbm,
                       srcv, dstv, rows, agg_sh, gsem):
    c = lax.axis_index("c")
    s = lax.axis_index("s")
    pltpu.sync_copy(zeros_hbm.at[pl.ds(s * ZR, ZR)], agg_sh.at[pl.ds(s * ZR, ZR)])
    wid = c * NS + s
    dbase = wid * CPW
    pltpu.sync_copy(srcs_hbm.at[pl.ds(dbase, DSTAGE)], srcv)
    pltpu.sync_copy(dsts_hbm.at[pl.ds(dbase, DSTAGE)], dstv)
    plsc.subcore_barrier()

    for b in range(NBUF - 1):
        pltpu.async_copy(m_hbm.at[srcv.at[b]], rows.at[b], gsem.at[b])

    def body(g, carry):
        buf = lax.rem(g, NBUF)
        drow = lax.rem(g, DSTAGE)
        pltpu.make_async_copy(m_hbm.at[srcv.at[drow]],
                              rows.at[buf], gsem.at[buf]).wait()

        @pl.when((drow == DSTAGE - 1) & (g + 1 < CPW))
        def _refresh_src():
            sstart = pl.multiple_of(dbase + g + 1, 8)
            pltpu.sync_copy(srcs_hbm.at[pl.ds(sstart, DSTAGE)], srcv)

        @pl.when(g + NBUF - 1 < CPW)
        def _prefetch():
            nb = lax.rem(g + NBUF - 1, NBUF)
            nrow = lax.rem(g + NBUF - 1, DSTAGE)
            pltpu.async_copy(m_hbm.at[srcv.at[nrow]],
                             rows.at[nb], gsem.at[nb])

        @pl.when((drow == 0) & (g > 0))
        def _refresh_dst():
            start = pl.multiple_of(dbase + g, 8)
            pltpu.sync_copy(dsts_hbm.at[pl.ds(start, DSTAGE)], dstv)

        pltpu.sync_copy(rows.at[buf], agg_sh.at[dstv.at[drow]], add=True)
        return carry

    lax.fori_loop(0, CPW, body, 0)
    plsc.subcore_barrier()
    pltpu.sync_copy(agg_sh.at[pl.ds(s * ZR, ZR)],
                    out_hbm.at[c, pl.ds(s * ZR, ZR)])


def _sigmoid(v):
    return 1.0 / (1.0 + jnp.exp(-v))


def _mm_body(x_ref, w_ref, o_ref):
    o_ref[...] = jnp.dot(x_ref[...], w_ref[...], precision=_PREC,
                         preferred_element_type=jnp.float32)


def _matmul(x, w):
    return pl.pallas_call(
        _mm_body,
        grid=(NB,),
        in_specs=[pl.BlockSpec((BN, H), lambda j: (j, 0)),
                  pl.BlockSpec((H, H), lambda j: (0, 0))],
        out_specs=pl.BlockSpec((BN, H), lambda j: (j, 0)),
        out_shape=jax.ShapeDtypeStruct((N, H), jnp.float32),
    )(x, w)


def _gru_math(p0, p1, h, wih, whh, bih, bhh):
    agg = p0[...] + p1[...]
    gi = jnp.dot(agg, wih[...], precision=_PREC,
                 preferred_element_type=jnp.float32) + bih[...]
    gh = jnp.dot(h[...], whh[...], precision=_PREC,
                 preferred_element_type=jnp.float32) + bhh[...]
    hv = h[...]
    r = _sigmoid(gi[:, 0:H] + gh[:, 0:H])
    z = _sigmoid(gi[:, H:2 * H] + gh[:, H:2 * H])
    n = jnp.tanh(gi[:, 2 * H:3 * H] + r * gh[:, 2 * H:3 * H])
    return (1.0 - z) * n + z * hv


def _gru_body(p0, p1, h, wih, whh, bih, bhh, wn, ho, mo):
    hn = _gru_math(p0, p1, h, wih, whh, bih, bhh)
    ho[...] = hn
    mo[...] = jnp.dot(hn, wn[...], precision=_PREC,
                      preferred_element_type=jnp.float32)


def _gru_next(p0, p1, h, wihT, whhT, bih2, bhh2, wnext):
    full = lambda r, c: pl.BlockSpec((r, c), lambda j: (0, 0))
    blk = pl.BlockSpec((BN, H), lambda j: (j, 0))
    return pl.pallas_call(
        _gru_body,
        grid=(NB,),
        in_specs=[blk, blk, blk, full(H, 3 * H), full(H, 3 * H),
                  full(1, 3 * H), full(1, 3 * H), full(H, H)],
        out_specs=[blk, blk],
        out_shape=[jax.ShapeDtypeStruct((N, H), jnp.float32),
                   jax.ShapeDtypeStruct((N, H), jnp.float32)],
    )(p0, p1, h, wihT, whhT, bih2, bhh2, wnext)


def _final_body(p0, p1, h, x, wih, whh, bih, bhh, wxr, whr, bh, bidx, out):
    j = pl.program_id(0)
    hn = _gru_math(p0, p1, h, wih, whh, bih, bhh)
    lg2 = x[...] * wxr[...] + hn * whr[...]
    rowlog = jnp.sum(lg2, axis=1, keepdims=True) + bh[...]
    bi = bidx[0]
    onehot = (lax.broadcasted_iota(jnp.int32, (G, BN), 0) == bi
              ).astype(jnp.float32)
    pp = jnp.dot(onehot, rowlog, precision=_PREC,
                 preferred_element_type=jnp.float32)

    @pl.when(j == 0)
    def _init():
        out[...] = jnp.zeros_like(out)

    out[...] += pp

    @pl.when(j == NB - 1)
    def _fin():
        out[...] = _sigmoid(out[...])


def _final(p0, p1, h, x, wihT, whhT, bih2, bhh2, wxr, whr, bh2, bidx3):
    full = lambda r, c: pl.BlockSpec((r, c), lambda j: (0, 0))
    blk = pl.BlockSpec((BN, H), lambda j: (j, 0))
    return pl.pallas_call(
        _final_body,
        grid=(NB,),
        in_specs=[blk, blk, blk, blk, full(H, 3 * H), full(H, 3 * H),
                  full(1, 3 * H), full(1, 3 * H), full(1, H), full(1, H),
                  full(1, 1), pl.BlockSpec((1, 1, BN), lambda j: (j, 0, 0))],
        out_specs=pl.BlockSpec((G, 1), lambda j: (0, 0)),
        out_shape=jax.ShapeDtypeStruct((G, 1), jnp.float32),
    )(p0, p1, h, x, wihT, whhT, bih2, bhh2, wxr, whr, bh2, bidx3)


def kernel(x, edge_index, batch_index, weight, w_ih, w_hh, b_ih, b_hh,
           w_head, b_head):
    src = edge_index[0]
    dst = edge_index[1]
    pad = EPAD - E
    src_p = jnp.concatenate([src, jnp.zeros((pad,), jnp.int32)])
    dst_p = jnp.concatenate([dst, jnp.full((pad,), N, jnp.int32)])
    srcs = src_p.reshape(NW * CPW, CHUNK)
    dsts = dst_p.reshape(NW * CPW, CHUNK)
    zeros_rows = jnp.zeros((NPAD, H), jnp.float32)
    wihT = w_ih.T
    whhT = w_hh.T
    bih2 = b_ih.reshape(1, 3 * H)
    bhh2 = b_hh.reshape(1, 3 * H)
    wxr = w_head[:, :H]
    whr = w_head[:, H:]
    bh2 = b_head.reshape(1, 1)
    bidx3 = batch_index.reshape(NB, 1, BN)

    h = x
    m = _matmul(x, weight[0])
    for i in range(L - 1):
        p = _sc_edge_aggregate(m, srcs, dsts, zeros_rows)
        h, m = _gru_next(p[0, :N], p[1, :N], h, wihT, whhT, bih2, bhh2,
                         weight[i + 1])
    p = _sc_edge_aggregate(m, srcs, dsts, zeros_rows)
    out = _final(p[0, :N], p[1, :N], h, x, wihT, whhT, bih2, bhh2, wxr, whr,
                 bh2, bidx3)
    return out[:, 0]


# speedup vs baseline: 1.0954x; 1.0954x over previous
import functools

import jax
import jax.numpy as jnp
from jax import lax
from jax.experimental import pallas as pl
from jax.experimental.pallas import tpu as pltpu
from jax.experimental.pallas import tpu_sc as plsc

N = 10000
E = 320000
H = 128
G = 64
L = 3

NC, NS = 2, 16
NW = NC * NS
CHUNK = 128
CPW0 = 120
CPW1 = 40
NBUF = 2
DSTAGE = 40
TOTCH = NS * (CPW0 + CPW1)
EPAD = TOTCH * CHUNK
NPAD = 10112
ZR = NPAD // NS

BN = 1000
NB = N // BN

_PREC = lax.Precision.DEFAULT

_sc_mesh = plsc.VectorSubcoreMesh(core_axis_name="c", subcore_axis_name="s")


@functools.partial(
    pl.kernel,
    out_type=jax.ShapeDtypeStruct((NC, NPAD, H), jnp.float32),
    mesh=_sc_mesh,
    scratch_types=[
        pltpu.VMEM((DSTAGE, CHUNK), jnp.int32),
        pltpu.VMEM((DSTAGE, CHUNK), jnp.int32),
        pltpu.VMEM((NBUF, CHUNK, H), jnp.float32),
        pltpu.VMEM_SHARED((NPAD, H), jnp.float32),
        pltpu.SemaphoreType.DMA((NBUF,)),
    ],
)
def _sc_edge_aggregate(m_hbm, srcs_hbm, dsts_hbm, zeros_hbm, out_hbm,
                       srcv, dstv, rows, agg_sh, gsem):
    c = lax.axis_index("c")
    s = lax.axis_index("s")
    pltpu.sync_copy(zeros_hbm.at[pl.ds(s * ZR, ZR)], agg_sh.at[pl.ds(s * ZR, ZR)])
    dbase = lax.select(c == 0, s * CPW0, NS * CPW0 + s * CPW1)
    cpw = lax.select(c == 0, CPW0, CPW1)
    pltpu.sync_copy(srcs_hbm.at[pl.ds(dbase, DSTAGE)], srcv)
    pltpu.sync_copy(dsts_hbm.at[pl.ds(dbase, DSTAGE)], dstv)
    plsc.subcore_barrier()

    for b in range(NBUF - 1):
        pltpu.async_copy(m_hbm.at[srcv.at[b]], rows.at[b], gsem.at[b])

    def body(g, carry):
        buf = lax.rem(g, NBUF)
        drow = lax.rem(g, DSTAGE)
        pltpu.make_async_copy(m_hbm.at[srcv.at[drow]],
                              rows.at[buf], gsem.at[buf]).wait()

        @pl.when((drow == DSTAGE - 1) & (g + 1 < cpw))
        def _refresh_src():
            sstart = pl.multiple_of(dbase + g + 1, 8)
            pltpu.sync_copy(srcs_hbm.at[pl.ds(sstart, DSTAGE)], srcv)

        @pl.when(g + NBUF - 1 < cpw)
        def _prefetch():
            nb = lax.rem(g + NBUF - 1, NBUF)
            nrow = lax.rem(g + NBUF - 1, DSTAGE)
            pltpu.async_copy(m_hbm.at[srcv.at[nrow]],
                             rows.at[nb], gsem.at[nb])

        @pl.when((drow == 0) & (g > 0))
        def _refresh_dst():
            start = pl.multiple_of(dbase + g, 8)
            pltpu.sync_copy(dsts_hbm.at[pl.ds(start, DSTAGE)], dstv)

        pltpu.sync_copy(rows.at[buf], agg_sh.at[dstv.at[drow]], add=True)
        return carry

    lax.fori_loop(0, cpw, body, 0)
    plsc.subcore_barrier()
    pltpu.sync_copy(agg_sh.at[pl.ds(s * ZR, ZR)],
                    out_hbm.at[c, pl.ds(s * ZR, ZR)])


def _sigmoid(v):
    return 1.0 / (1.0 + jnp.exp(-v))


def _mm_body(x_ref, w_ref, o_ref):
    o_ref[...] = jnp.dot(x_ref[...], w_ref[...], precision=_PREC,
                         preferred_element_type=jnp.float32)


def _matmul(x, w):
    return pl.pallas_call(
        _mm_body,
        grid=(NB,),
        in_specs=[pl.BlockSpec((BN, H), lambda j: (j, 0)),
                  pl.BlockSpec((H, H), lambda j: (0, 0))],
        out_specs=pl.BlockSpec((BN, H), lambda j: (j, 0)),
        out_shape=jax.ShapeDtypeStruct((N, H), jnp.float32),
    )(x, w)


def _gru_math(p0, p1, h, wih, whh, bih, bhh):
    agg = p0[...] + p1[...]
    gi = jnp.dot(agg, wih[...], precision=_PREC,
                 preferred_element_type=jnp.float32) + bih[...]
    gh = jnp.dot(h[...], whh[...], precision=_PREC,
                 preferred_element_type=jnp.float32) + bhh[...]
    hv = h[...]
    r = _sigmoid(gi[:, 0:H] + gh[:, 0:H])
    z = _sigmoid(gi[:, H:2 * H] + gh[:, H:2 * H])
    n = jnp.tanh(gi[:, 2 * H:3 * H] + r * gh[:, 2 * H:3 * H])
    return (1.0 - z) * n + z * hv


def _gru_body(p0, p1, h, wih, whh, bih, bhh, wn, ho, mo):
    hn = _gru_math(p0, p1, h, wih, whh, bih, bhh)
    ho[...] = hn
    mo[...] = jnp.dot(hn, wn[...], precision=_PREC,
                      preferred_element_type=jnp.float32)


def _gru_next(p0, p1, h, wihT, whhT, bih2, bhh2, wnext):
    full = lambda r, c: pl.BlockSpec((r, c), lambda j: (0, 0))
    blk = pl.BlockSpec((BN, H), lambda j: (j, 0))
    return pl.pallas_call(
        _gru_body,
        grid=(NB,),
        in_specs=[blk, blk, blk, full(H, 3 * H), full(H, 3 * H),
                  full(1, 3 * H), full(1, 3 * H), full(H, H)],
        out_specs=[blk, blk],
        out_shape=[jax.ShapeDtypeStruct((N, H), jnp.float32),
                   jax.ShapeDtypeStruct((N, H), jnp.float32)],
    )(p0, p1, h, wihT, whhT, bih2, bhh2, wnext)


def _final_body(p0, p1, h, x, wih, whh, bih, bhh, wxr, whr, bh, bidx, out):
    j = pl.program_id(0)
    hn = _gru_math(p0, p1, h, wih, whh, bih, bhh)
    lg2 = x[...] * wxr[...] + hn * whr[...]
    rowlog = jnp.sum(lg2, axis=1, keepdims=True) + bh[...]
    bi = bidx[0]
    onehot = (lax.broadcasted_iota(jnp.int32, (G, BN), 0) == bi
              ).astype(jnp.float32)
    pp = jnp.dot(onehot, rowlog, precision=_PREC,
                 preferred_element_type=jnp.float32)

    @pl.when(j == 0)
    def _init():
        out[...] = jnp.zeros_like(out)

    out[...] += pp

    @pl.when(j == NB - 1)
    def _fin():
        out[...] = _sigmoid(out[...])


def _final(p0, p1, h, x, wihT, whhT, bih2, bhh2, wxr, whr, bh2, bidx3):
    full = lambda r, c: pl.BlockSpec((r, c), lambda j: (0, 0))
    blk = pl.BlockSpec((BN, H), lambda j: (j, 0))
    return pl.pallas_call(
        _final_body,
        grid=(NB,),
        in_specs=[blk, blk, blk, blk, full(H, 3 * H), full(H, 3 * H),
                  full(1, 3 * H), full(1, 3 * H), full(1, H), full(1, H),
                  full(1, 1), pl.BlockSpec((1, 1, BN), lambda j: (j, 0, 0))],
        out_specs=pl.BlockSpec((G, 1), lambda j: (0, 0)),
        out_shape=jax.ShapeDtypeStruct((G, 1), jnp.float32),
    )(p0, p1, h, x, wihT, whhT, bih2, bhh2, wxr, whr, bh2, bidx3)


def kernel(x, edge_index, batch_index, weight, w_ih, w_hh, b_ih, b_hh,
           w_head, b_head):
    src = edge_index[0]
    dst = edge_index[1]
    pad = EPAD - E
    src_p = jnp.concatenate([src, jnp.zeros((pad,), jnp.int32)])
    dst_p = jnp.concatenate([dst, jnp.full((pad,), N, jnp.int32)])
    srcs = src_p.reshape(TOTCH, CHUNK)
    dsts = dst_p.reshape(TOTCH, CHUNK)
    zeros_rows = jnp.zeros((NPAD, H), jnp.float32)
    wihT = w_ih.T
    whhT = w_hh.T
    bih2 = b_ih.reshape(1, 3 * H)
    bhh2 = b_hh.reshape(1, 3 * H)
    wxr = w_head[:, :H]
    whr = w_head[:, H:]
    bh2 = b_head.reshape(1, 1)
    bidx3 = batch_index.reshape(NB, 1, BN)

    h = x
    m = _matmul(x, weight[0])
    for i in range(L - 1):
        p = _sc_edge_aggregate(m, srcs, dsts, zeros_rows)
        h, m = _gru_next(p[0, :N], p[1, :N], h, wihT, whhT, bih2, bhh2,
                         weight[i + 1])
    p = _sc_edge_aggregate(m, srcs, dsts, zeros_rows)
    out = _final(p[0, :N], p[1, :N], h, x, wihT, whhT, bih2, bhh2, wxr, whr,
                 bh2, bidx3)
    return out[:, 0]
